# Initial kernel scaffold; baseline (speedup 1.0000x reference)
import jax
import jax.numpy as jnp
from jax.experimental import pallas as pl


def kernel(positions, hidden_states, Wq, bq, Wk, bk, Wv, bv, Wo, ln1, ln2, Wr, Weg, Weu, Wed, Wsg, Wsu, Wsd, Wse):
    raise NotImplementedError("write your pallas kernel here")


# speedup vs baseline: 1.1313x; 1.1313x over previous
import functools
import jax
import jax.numpy as jnp
from jax.experimental import pallas as pl
from jax.experimental.pallas import tpu as pltpu

HIDDEN = 1024
N_HEADS = 16
N_KV_HEADS = 4
HEAD_DIM = 64
N_EXPERTS = 8
TOP_K = 2
MOE_FF = 1408
SHARED_FF = 2816
EPS = 1e-6
ROPE_BASE = 1000000.0
T = 2048

BT = 256


def _dot(a, b):
    return jax.lax.dot_general(a, b, (((1,), (0,)), ((), ())),
                               preferred_element_type=jnp.float32)


def _dot_t(a, b):
    return jax.lax.dot_general(a, b, (((1,), (1,)), ((), ())),
                               preferred_element_type=jnp.float32)


def _rms(x, scale):
    var = jnp.mean(jnp.square(x), axis=-1, keepdims=True)
    return x * jax.lax.rsqrt(var + EPS) * scale


def _rope_2d(pos, x, n_heads):
    half = HEAD_DIM // 2
    x3 = x.reshape(x.shape[0], n_heads, HEAD_DIM)
    inv_freq = jnp.exp(
        jnp.arange(0, half, dtype=jnp.int32).astype(jnp.float32)
        * (-jnp.log(ROPE_BASE) / half))
    freqs = pos[:, None] * inv_freq[None, :]
    cos = jnp.cos(freqs)[:, None, :]
    sin = jnp.sin(freqs)[:, None, :]
    x1 = x3[..., :half]
    x2 = x3[..., half:]
    r = jnp.concatenate([x1 * cos - x2 * sin, x2 * cos + x1 * sin], axis=-1)
    return r.reshape(x.shape[0], n_heads * HEAD_DIM)


def _k1_body(pos_ref, h_ref, wq_ref, bq_ref, wk_ref, bk_ref, wv_ref, bv_ref,
             ln1_ref, q_ref, k_ref, v_ref):
    h = _rms(h_ref[...], ln1_ref[...])
    pos = pos_ref[0, 0, :].astype(jnp.float32)
    q = _dot(h, wq_ref[...]) + bq_ref[...]
    k = _dot(h, wk_ref[...]) + bk_ref[...]
    v = _dot(h, wv_ref[...]) + bv_ref[...]
    q_ref[...] = _rope_2d(pos, q, N_HEADS)
    k_ref[...] = _rope_2d(pos, k, N_KV_HEADS)
    v_ref[...] = v


def _pre_attn(positions, hidden_states, Wq, bq, Wk, bk, Wv, bv, ln1):
    pos3 = positions.reshape(T // BT, 1, BT)
    return pl.pallas_call(
        _k1_body,
        grid=(T // BT,),
        in_specs=[
            pl.BlockSpec((1, 1, BT), lambda i: (i, 0, 0)),
            pl.BlockSpec((BT, HIDDEN), lambda i: (i, 0)),
            pl.BlockSpec((HIDDEN, N_HEADS * HEAD_DIM), lambda i: (0, 0)),
            pl.BlockSpec((1, N_HEADS * HEAD_DIM), lambda i: (0, 0)),
            pl.BlockSpec((HIDDEN, N_KV_HEADS * HEAD_DIM), lambda i: (0, 0)),
            pl.BlockSpec((1, N_KV_HEADS * HEAD_DIM), lambda i: (0, 0)),
            pl.BlockSpec((HIDDEN, N_KV_HEADS * HEAD_DIM), lambda i: (0, 0)),
            pl.BlockSpec((1, N_KV_HEADS * HEAD_DIM), lambda i: (0, 0)),
            pl.BlockSpec((1, HIDDEN), lambda i: (0, 0)),
        ],
        out_specs=[
            pl.BlockSpec((BT, N_HEADS * HEAD_DIM), lambda i: (i, 0)),
            pl.BlockSpec((BT, N_KV_HEADS * HEAD_DIM), lambda i: (i, 0)),
            pl.BlockSpec((BT, N_KV_HEADS * HEAD_DIM), lambda i: (i, 0)),
        ],
        out_shape=[
            jax.ShapeDtypeStruct((T, N_HEADS * HEAD_DIM), jnp.float32),
            jax.ShapeDtypeStruct((T, N_KV_HEADS * HEAD_DIM), jnp.float32),
            jax.ShapeDtypeStruct((T, N_KV_HEADS * HEAD_DIM), jnp.float32),
        ],
    )(pos3, hidden_states, Wq, bq.reshape(1, -1), Wk, bk.reshape(1, -1),
      Wv, bv.reshape(1, -1), ln1.reshape(1, -1))


def _k2_body(q_ref, k_ref, v_ref, o_ref):
    i = pl.program_id(1)
    q = q_ref[0]
    k = k_ref[0]
    v = v_ref[0]
    scores = _dot_t(q, k) * (HEAD_DIM ** -0.5)
    q_pos = i * BT + jax.lax.broadcasted_iota(jnp.int32, scores.shape, 0)
    k_pos = jax.lax.broadcasted_iota(jnp.int32, scores.shape, 1)
    scores = jnp.where(k_pos <= q_pos, scores, jnp.float32(-1e9))
    m = jnp.max(scores, axis=-1, keepdims=True)
    p = jnp.exp(scores - m)
    p = p / jnp.sum(p, axis=-1, keepdims=True)
    o_ref[0] = _dot(p, v)


def _attention(q, k, v):
    rep = N_HEADS // N_KV_HEADS
    return pl.pallas_call(
        _k2_body,
        grid=(N_HEADS, T // BT),
        in_specs=[
            pl.BlockSpec((1, BT, HEAD_DIM), lambda h, i: (h, i, 0)),
            pl.BlockSpec((1, T, HEAD_DIM), lambda h, i: (h // rep, 0, 0)),
            pl.BlockSpec((1, T, HEAD_DIM), lambda h, i: (h // rep, 0, 0)),
        ],
        out_specs=pl.BlockSpec((1, BT, HEAD_DIM), lambda h, i: (h, i, 0)),
        out_shape=jax.ShapeDtypeStruct((N_HEADS, T, HEAD_DIM), jnp.float32),
    )(q, k, v)


def _k3_body(attn_ref, wo_ref, res_ref, ln2_ref, res2_ref, h2_ref):
    hidden = _dot(attn_ref[...], wo_ref[...]) + res_ref[...]
    res2_ref[...] = hidden
    h2_ref[...] = _rms(hidden, ln2_ref[...])


def _post_attn(attn, Wo, residual, ln2):
    return pl.pallas_call(
        _k3_body,
        grid=(T // BT,),
        in_specs=[
            pl.BlockSpec((BT, N_HEADS * HEAD_DIM), lambda i: (i, 0)),
            pl.BlockSpec((N_HEADS * HEAD_DIM, HIDDEN), lambda i: (0, 0)),
            pl.BlockSpec((BT, HIDDEN), lambda i: (i, 0)),
            pl.BlockSpec((1, HIDDEN), lambda i: (0, 0)),
        ],
        out_specs=[
            pl.BlockSpec((BT, HIDDEN), lambda i: (i, 0)),
            pl.BlockSpec((BT, HIDDEN), lambda i: (i, 0)),
        ],
        out_shape=[
            jax.ShapeDtypeStruct((T, HIDDEN), jnp.float32),
            jax.ShapeDtypeStruct((T, HIDDEN), jnp.float32),
        ],
    )(attn, Wo, residual, ln2.reshape(1, -1))


def _k4_body(h2_ref, wsg_ref, wsu_ref, wsd_ref, wse_ref, out_ref):
    h2 = h2_ref[...]
    g = _dot(h2, wsg_ref[...])
    u = _dot(h2, wsu_ref[...])
    y = _dot(g * jax.lax.logistic(g) * u, wsd_ref[...])
    gate = jax.lax.logistic(_dot(h2, wse_ref[...]))
    out_ref[...] = gate * y


def _shared_expert(h2, Wsg, Wsu, Wsd, Wse):
    return pl.pallas_call(
        _k4_body,
        grid=(T // BT,),
        in_specs=[
            pl.BlockSpec((BT, HIDDEN), lambda i: (i, 0)),
            pl.BlockSpec((HIDDEN, SHARED_FF), lambda i: (0, 0)),
            pl.BlockSpec((HIDDEN, SHARED_FF), lambda i: (0, 0)),
            pl.BlockSpec((SHARED_FF, HIDDEN), lambda i: (0, 0)),
            pl.BlockSpec((HIDDEN, 1), lambda i: (0, 0)),
        ],
        out_specs=pl.BlockSpec((BT, HIDDEN), lambda i: (i, 0)),
        out_shape=jax.ShapeDtypeStruct((T, HIDDEN), jnp.float32),
    )(h2, Wsg, Wsu, Wsd, Wse)


def _k5_body(h2_ref, wr_ref, comb_ref):
    logits = _dot(h2_ref[...], wr_ref[...])
    m = jnp.max(logits, axis=-1, keepdims=True)
    e = jnp.exp(logits - m)
    probs = e / jnp.sum(e, axis=-1, keepdims=True)
    lane = jax.lax.broadcasted_iota(jnp.int32, probs.shape, 1)
    m1 = jnp.max(probs, axis=-1, keepdims=True)
    is1 = (probs == m1)
    i1 = jnp.min(jnp.where(is1, lane, N_EXPERTS), axis=-1, keepdims=True)
    oh1 = (lane == i1)
    p2 = jnp.where(oh1, -jnp.float32(1.0), probs)
    m2 = jnp.max(p2, axis=-1, keepdims=True)
    is2 = (p2 == m2)
    i2 = jnp.min(jnp.where(is2, lane, N_EXPERTS), axis=-1, keepdims=True)
    oh2 = (lane == i2)
    denom = m1 + m2
    comb_ref[...] = jnp.where(oh1, m1 / denom,
                              jnp.where(oh2, m2 / denom, 0.0))


def _router(h2, Wr):
    return pl.pallas_call(
        _k5_body,
        grid=(1,),
        in_specs=[
            pl.BlockSpec((T, HIDDEN), lambda i: (0, 0)),
            pl.BlockSpec((HIDDEN, N_EXPERTS), lambda i: (0, 0)),
        ],
        out_specs=pl.BlockSpec((T, N_EXPERTS), lambda i: (0, 0)),
        out_shape=jax.ShapeDtypeStruct((T, N_EXPERTS), jnp.float32),
    )(h2, Wr)


def _k6_body(h2_ref, weg_ref, weu_ref, wed_ref, comb_ref, out_ref):
    e = pl.program_id(0)
    t = pl.program_id(1)
    h2 = h2_ref[...]
    g = _dot(h2, weg_ref[0])
    u = _dot(h2, weu_ref[0])
    y = _dot(g * jax.lax.logistic(g) * u, wed_ref[0])
    lane = jax.lax.broadcasted_iota(jnp.int32, comb_ref.shape, 1)
    w = jnp.sum(jnp.where(lane == e, comb_ref[...], 0.0), axis=-1,
                keepdims=True)
    sl = (pl.ds(t * BT, BT), slice(None))

    @pl.when(e == 0)
    def _():
        out_ref[sl] = w * y

    @pl.when(e != 0)
    def _():
        out_ref[sl] = out_ref[sl] + w * y


def _moe_dense(h2, Weg, Weu, Wed, combine):
    return pl.pallas_call(
        _k6_body,
        grid=(N_EXPERTS, T // BT),
        in_specs=[
            pl.BlockSpec((BT, HIDDEN), lambda e, t: (t, 0)),
            pl.BlockSpec((1, HIDDEN, MOE_FF), lambda e, t: (e, 0, 0)),
            pl.BlockSpec((1, HIDDEN, MOE_FF), lambda e, t: (e, 0, 0)),
            pl.BlockSpec((1, MOE_FF, HIDDEN), lambda e, t: (e, 0, 0)),
            pl.BlockSpec((BT, N_EXPERTS), lambda e, t: (t, 0)),
        ],
        out_specs=pl.BlockSpec((T, HIDDEN), lambda e, t: (0, 0)),
        out_shape=jax.ShapeDtypeStruct((T, HIDDEN), jnp.float32),
    )(h2, Weg, Weu, Wed, combine)


def _k7_body(moe_ref, shared_ref, res2_ref, out_ref):
    out_ref[...] = moe_ref[...] + shared_ref[...] + res2_ref[...]


def _final(moe, shared, res2):
    return pl.pallas_call(
        _k7_body,
        grid=(T // BT,),
        in_specs=[pl.BlockSpec((BT, HIDDEN), lambda i: (i, 0))] * 3,
        out_specs=pl.BlockSpec((BT, HIDDEN), lambda i: (i, 0)),
        out_shape=jax.ShapeDtypeStruct((T, HIDDEN), jnp.float32),
    )(moe, shared, res2)


@jax.jit
def kernel(positions, hidden_states, Wq, bq, Wk, bk, Wv, bv, Wo, ln1, ln2,
           Wr, Weg, Weu, Wed, Wsg, Wsu, Wsd, Wse):
    q, k, v = _pre_attn(positions, hidden_states, Wq, bq, Wk, bk, Wv, bv, ln1)
    q3 = q.reshape(T, N_HEADS, HEAD_DIM).transpose(1, 0, 2)
    k3 = k.reshape(T, N_KV_HEADS, HEAD_DIM).transpose(1, 0, 2)
    v3 = v.reshape(T, N_KV_HEADS, HEAD_DIM).transpose(1, 0, 2)
    attn3 = _attention(q3, k3, v3)
    attn = attn3.transpose(1, 0, 2).reshape(T, N_HEADS * HEAD_DIM)
    res2, h2 = _post_attn(attn, Wo, hidden_states, ln2)
    shared = _shared_expert(h2, Wsg, Wsu, Wsd, Wse)
    combine = _router(h2, Wr)
    moe = _moe_dense(h2, Weg, Weu, Wed, combine)
    return _final(moe, shared, res2)


# speedup vs baseline: 1.2354x; 1.0920x over previous
import functools
import jax
import jax.numpy as jnp
from jax.experimental import pallas as pl
from jax.experimental.pallas import tpu as pltpu

HIDDEN = 1024
N_HEADS = 16
N_KV_HEADS = 4
HEAD_DIM = 64
N_EXPERTS = 8
TOP_K = 2
MOE_FF = 1408
SHARED_FF = 2816
EPS = 1e-6
ROPE_BASE = 1000000.0
T = 2048

BT = 256


def _dot(a, b):
    return jax.lax.dot_general(a.astype(jnp.bfloat16), b.astype(jnp.bfloat16),
                               (((1,), (0,)), ((), ())),
                               preferred_element_type=jnp.float32)


def _dot_t(a, b):
    return jax.lax.dot_general(a.astype(jnp.bfloat16), b.astype(jnp.bfloat16),
                               (((1,), (1,)), ((), ())),
                               preferred_element_type=jnp.float32)


def _rms(x, scale):
    var = jnp.mean(jnp.square(x), axis=-1, keepdims=True)
    return x * jax.lax.rsqrt(var + EPS) * scale


def _rope_2d(pos, x, n_heads):
    half = HEAD_DIM // 2
    x3 = x.reshape(x.shape[0], n_heads, HEAD_DIM)
    inv_freq = jnp.exp(
        jnp.arange(0, half, dtype=jnp.int32).astype(jnp.float32)
        * (-jnp.log(ROPE_BASE) / half))
    freqs = pos[:, None] * inv_freq[None, :]
    cos = jnp.cos(freqs)[:, None, :]
    sin = jnp.sin(freqs)[:, None, :]
    x1 = x3[..., :half]
    x2 = x3[..., half:]
    r = jnp.concatenate([x1 * cos - x2 * sin, x2 * cos + x1 * sin], axis=-1)
    return r.reshape(x.shape[0], n_heads * HEAD_DIM)


def _k1_body(pos_ref, h_ref, wq_ref, bq_ref, wk_ref, bk_ref, wv_ref, bv_ref,
             ln1_ref, q_ref, k_ref, v_ref):
    h = _rms(h_ref[...], ln1_ref[...])
    pos = pos_ref[0, 0, :].astype(jnp.float32)
    q = _dot(h, wq_ref[...]) + bq_ref[...]
    k = _dot(h, wk_ref[...]) + bk_ref[...]
    v = _dot(h, wv_ref[...]) + bv_ref[...]
    q_ref[...] = _rope_2d(pos, q, N_HEADS)
    k_ref[...] = _rope_2d(pos, k, N_KV_HEADS)
    v_ref[...] = v


def _pre_attn(positions, hidden_states, Wq, bq, Wk, bk, Wv, bv, ln1):
    pos3 = positions.reshape(T // BT, 1, BT)
    return pl.pallas_call(
        _k1_body,
        grid=(T // BT,),
        in_specs=[
            pl.BlockSpec((1, 1, BT), lambda i: (i, 0, 0)),
            pl.BlockSpec((BT, HIDDEN), lambda i: (i, 0)),
            pl.BlockSpec((HIDDEN, N_HEADS * HEAD_DIM), lambda i: (0, 0)),
            pl.BlockSpec((1, N_HEADS * HEAD_DIM), lambda i: (0, 0)),
            pl.BlockSpec((HIDDEN, N_KV_HEADS * HEAD_DIM), lambda i: (0, 0)),
            pl.BlockSpec((1, N_KV_HEADS * HEAD_DIM), lambda i: (0, 0)),
            pl.BlockSpec((HIDDEN, N_KV_HEADS * HEAD_DIM), lambda i: (0, 0)),
            pl.BlockSpec((1, N_KV_HEADS * HEAD_DIM), lambda i: (0, 0)),
            pl.BlockSpec((1, HIDDEN), lambda i: (0, 0)),
        ],
        out_specs=[
            pl.BlockSpec((BT, N_HEADS * HEAD_DIM), lambda i: (i, 0)),
            pl.BlockSpec((BT, N_KV_HEADS * HEAD_DIM), lambda i: (i, 0)),
            pl.BlockSpec((BT, N_KV_HEADS * HEAD_DIM), lambda i: (i, 0)),
        ],
        out_shape=[
            jax.ShapeDtypeStruct((T, N_HEADS * HEAD_DIM), jnp.float32),
            jax.ShapeDtypeStruct((T, N_KV_HEADS * HEAD_DIM), jnp.float32),
            jax.ShapeDtypeStruct((T, N_KV_HEADS * HEAD_DIM), jnp.float32),
        ],
    )(pos3, hidden_states, Wq, bq.reshape(1, -1), Wk, bk.reshape(1, -1),
      Wv, bv.reshape(1, -1), ln1.reshape(1, -1))


def _k2_body(q_ref, k_ref, v_ref, o_ref):
    i = pl.program_id(1)
    q = q_ref[0]
    k = k_ref[0]
    v = v_ref[0]
    scores = _dot_t(q, k) * (HEAD_DIM ** -0.5)
    q_pos = i * BT + jax.lax.broadcasted_iota(jnp.int32, scores.shape, 0)
    k_pos = jax.lax.broadcasted_iota(jnp.int32, scores.shape, 1)
    scores = jnp.where(k_pos <= q_pos, scores, jnp.float32(-1e9))
    m = jnp.max(scores, axis=-1, keepdims=True)
    p = jnp.exp(scores - m)
    p = p / jnp.sum(p, axis=-1, keepdims=True)
    o_ref[0] = _dot(p, v)


def _attention(q, k, v):
    rep = N_HEADS // N_KV_HEADS
    return pl.pallas_call(
        _k2_body,
        grid=(N_HEADS, T // BT),
        in_specs=[
            pl.BlockSpec((1, BT, HEAD_DIM), lambda h, i: (h, i, 0)),
            pl.BlockSpec((1, T, HEAD_DIM), lambda h, i: (h // rep, 0, 0)),
            pl.BlockSpec((1, T, HEAD_DIM), lambda h, i: (h // rep, 0, 0)),
        ],
        out_specs=pl.BlockSpec((1, BT, HEAD_DIM), lambda h, i: (h, i, 0)),
        out_shape=jax.ShapeDtypeStruct((N_HEADS, T, HEAD_DIM), jnp.float32),
    )(q, k, v)


def _k3_body(attn_ref, wo_ref, res_ref, ln2_ref, res2_ref, h2_ref):
    hidden = _dot(attn_ref[...], wo_ref[...]) + res_ref[...]
    res2_ref[...] = hidden
    h2_ref[...] = _rms(hidden, ln2_ref[...])


def _post_attn(attn, Wo, residual, ln2):
    return pl.pallas_call(
        _k3_body,
        grid=(T // BT,),
        in_specs=[
            pl.BlockSpec((BT, N_HEADS * HEAD_DIM), lambda i: (i, 0)),
            pl.BlockSpec((N_HEADS * HEAD_DIM, HIDDEN), lambda i: (0, 0)),
            pl.BlockSpec((BT, HIDDEN), lambda i: (i, 0)),
            pl.BlockSpec((1, HIDDEN), lambda i: (0, 0)),
        ],
        out_specs=[
            pl.BlockSpec((BT, HIDDEN), lambda i: (i, 0)),
            pl.BlockSpec((BT, HIDDEN), lambda i: (i, 0)),
        ],
        out_shape=[
            jax.ShapeDtypeStruct((T, HIDDEN), jnp.float32),
            jax.ShapeDtypeStruct((T, HIDDEN), jnp.float32),
        ],
    )(attn, Wo, residual, ln2.reshape(1, -1))


def _k4_body(h2_ref, wsg_ref, wsu_ref, wsd_ref, wse_ref, out_ref):
    h2 = h2_ref[...]
    g = _dot(h2, wsg_ref[...])
    u = _dot(h2, wsu_ref[...])
    y = _dot(g * jax.lax.logistic(g) * u, wsd_ref[...])
    gate = jax.lax.logistic(_dot(h2, wse_ref[...]))
    out_ref[...] = gate * y


def _shared_expert(h2, Wsg, Wsu, Wsd, Wse):
    return pl.pallas_call(
        _k4_body,
        grid=(T // BT,),
        in_specs=[
            pl.BlockSpec((BT, HIDDEN), lambda i: (i, 0)),
            pl.BlockSpec((HIDDEN, SHARED_FF), lambda i: (0, 0)),
            pl.BlockSpec((HIDDEN, SHARED_FF), lambda i: (0, 0)),
            pl.BlockSpec((SHARED_FF, HIDDEN), lambda i: (0, 0)),
            pl.BlockSpec((HIDDEN, 1), lambda i: (0, 0)),
        ],
        out_specs=pl.BlockSpec((BT, HIDDEN), lambda i: (i, 0)),
        out_shape=jax.ShapeDtypeStruct((T, HIDDEN), jnp.float32),
    )(h2, Wsg, Wsu, Wsd, Wse)


def _k5_body(h2_ref, wr_ref, comb_ref):
    logits = _dot(h2_ref[...], wr_ref[...])
    m = jnp.max(logits, axis=-1, keepdims=True)
    e = jnp.exp(logits - m)
    probs = e / jnp.sum(e, axis=-1, keepdims=True)
    lane = jax.lax.broadcasted_iota(jnp.int32, probs.shape, 1)
    m1 = jnp.max(probs, axis=-1, keepdims=True)
    is1 = (probs == m1)
    i1 = jnp.min(jnp.where(is1, lane, N_EXPERTS), axis=-1, keepdims=True)
    oh1 = (lane == i1)
    p2 = jnp.where(oh1, -jnp.float32(1.0), probs)
    m2 = jnp.max(p2, axis=-1, keepdims=True)
    is2 = (p2 == m2)
    i2 = jnp.min(jnp.where(is2, lane, N_EXPERTS), axis=-1, keepdims=True)
    oh2 = (lane == i2)
    denom = m1 + m2
    comb_ref[...] = jnp.where(oh1, m1 / denom,
                              jnp.where(oh2, m2 / denom, 0.0))


def _router(h2, Wr):
    return pl.pallas_call(
        _k5_body,
        grid=(1,),
        in_specs=[
            pl.BlockSpec((T, HIDDEN), lambda i: (0, 0)),
            pl.BlockSpec((HIDDEN, N_EXPERTS), lambda i: (0, 0)),
        ],
        out_specs=pl.BlockSpec((T, N_EXPERTS), lambda i: (0, 0)),
        out_shape=jax.ShapeDtypeStruct((T, N_EXPERTS), jnp.float32),
    )(h2, Wr)


def _k6_body(h2_ref, weg_ref, weu_ref, wed_ref, comb_ref, out_ref):
    e = pl.program_id(0)
    t = pl.program_id(1)
    h2 = h2_ref[...]
    g = _dot(h2, weg_ref[0])
    u = _dot(h2, weu_ref[0])
    y = _dot(g * jax.lax.logistic(g) * u, wed_ref[0])
    lane = jax.lax.broadcasted_iota(jnp.int32, comb_ref.shape, 1)
    w = jnp.sum(jnp.where(lane == e, comb_ref[...], 0.0), axis=-1,
                keepdims=True)
    sl = (pl.ds(t * BT, BT), slice(None))

    @pl.when(e == 0)
    def _():
        out_ref[sl] = w * y

    @pl.when(e != 0)
    def _():
        out_ref[sl] = out_ref[sl] + w * y


def _moe_dense(h2, Weg, Weu, Wed, combine):
    return pl.pallas_call(
        _k6_body,
        grid=(N_EXPERTS, T // BT),
        in_specs=[
            pl.BlockSpec((BT, HIDDEN), lambda e, t: (t, 0)),
            pl.BlockSpec((1, HIDDEN, MOE_FF), lambda e, t: (e, 0, 0)),
            pl.BlockSpec((1, HIDDEN, MOE_FF), lambda e, t: (e, 0, 0)),
            pl.BlockSpec((1, MOE_FF, HIDDEN), lambda e, t: (e, 0, 0)),
            pl.BlockSpec((BT, N_EXPERTS), lambda e, t: (t, 0)),
        ],
        out_specs=pl.BlockSpec((T, HIDDEN), lambda e, t: (0, 0)),
        out_shape=jax.ShapeDtypeStruct((T, HIDDEN), jnp.float32),
    )(h2, Weg, Weu, Wed, combine)


def _k7_body(moe_ref, shared_ref, res2_ref, out_ref):
    out_ref[...] = moe_ref[...] + shared_ref[...] + res2_ref[...]


def _final(moe, shared, res2):
    return pl.pallas_call(
        _k7_body,
        grid=(T // BT,),
        in_specs=[pl.BlockSpec((BT, HIDDEN), lambda i: (i, 0))] * 3,
        out_specs=pl.BlockSpec((BT, HIDDEN), lambda i: (i, 0)),
        out_shape=jax.ShapeDtypeStruct((T, HIDDEN), jnp.float32),
    )(moe, shared, res2)


@jax.jit
def kernel(positions, hidden_states, Wq, bq, Wk, bk, Wv, bv, Wo, ln1, ln2,
           Wr, Weg, Weu, Wed, Wsg, Wsu, Wsd, Wse):
    q, k, v = _pre_attn(positions, hidden_states, Wq, bq, Wk, bk, Wv, bv, ln1)
    q3 = q.reshape(T, N_HEADS, HEAD_DIM).transpose(1, 0, 2)
    k3 = k.reshape(T, N_KV_HEADS, HEAD_DIM).transpose(1, 0, 2)
    v3 = v.reshape(T, N_KV_HEADS, HEAD_DIM).transpose(1, 0, 2)
    attn3 = _attention(q3, k3, v3)
    attn = attn3.transpose(1, 0, 2).reshape(T, N_HEADS * HEAD_DIM)
    res2, h2 = _post_attn(attn, Wo, hidden_states, ln2)
    shared = _shared_expert(h2, Wsg, Wsu, Wsd, Wse)
    combine = _router(h2, Wr)
    moe = _moe_dense(h2, Weg, Weu, Wed, combine)
    return _final(moe, shared, res2)
